# Initial kernel scaffold; baseline (speedup 1.0000x reference)
import jax
import jax.numpy as jnp
from jax.experimental import pallas as pl


def kernel(msg, h, W):
    raise NotImplementedError("write your pallas kernel here")


# speedup vs baseline: 2.0570x; 2.0570x over previous
import jax
import jax.numpy as jnp
from jax.experimental import pallas as pl
from jax.experimental.pallas import tpu as pltpu

B_BLOCK = 1000


def _body(msg_ref, h_ref, wh_ref, wm_ref, out_ref):
    K = msg_ref.shape[1]
    msum = jnp.sum(msg_ref[...], axis=1)
    hk = h_ref[...] * jnp.float32(K)
    out_ref[...] = jnp.dot(
        hk, wh_ref[...], preferred_element_type=jnp.float32
    ) + jnp.dot(msum, wm_ref[...], preferred_element_type=jnp.float32)


@jax.jit
def kernel(msg, h, W):
    B, K, d_msg = msg.shape
    d_h = h.shape[-1]
    Wh = W[:d_h]
    Wm = W[d_h:]
    grid = (B // B_BLOCK,)
    return pl.pallas_call(
        _body,
        grid=grid,
        in_specs=[
            pl.BlockSpec((B_BLOCK, K, d_msg), lambda i: (i, 0, 0)),
            pl.BlockSpec((B_BLOCK, d_h), lambda i: (i, 0)),
            pl.BlockSpec((d_h, W.shape[1]), lambda i: (0, 0)),
            pl.BlockSpec((d_msg, W.shape[1]), lambda i: (0, 0)),
        ],
        out_specs=pl.BlockSpec((B_BLOCK, W.shape[1]), lambda i: (i, 0)),
        out_shape=jax.ShapeDtypeStruct((B, W.shape[1]), jnp.float32),
    )(msg, h, Wh, Wm)
